# initial kernel scaffold (unmeasured)
import jax
import jax.numpy as jnp
from jax import lax
from jax.experimental import pallas as pl
from jax.experimental.pallas import tpu as pltpu


def kernel(
    x,
):
    def body(*refs):
        pass

    out_shape = jax.ShapeDtypeStruct(..., jnp.float32)
    return pl.pallas_call(body, out_shape=out_shape)(...)


# baseline (device time: 308820 ns/iter reference)
import jax
import jax.numpy as jnp
from jax import lax
from jax.experimental import pallas as pl
from jax.experimental.pallas import tpu as pltpu

N_DEV = 4


def kernel(x):
    m, n = x.shape
    mc = m // N_DEV

    def body(x_ref, out_ref, rs_ref, ag_ref,
             rs_send_sems, rs_recv_sems, ag_send_sems, ag_recv_sems):
        my = lax.axis_index("i")
        left = lax.rem(my + N_DEV - 1, N_DEV)
        right = lax.rem(my + 1, N_DEV)

        barrier_sem = pltpu.get_barrier_semaphore()
        for nbr in (left, right):
            pl.semaphore_signal(
                barrier_sem, inc=1,
                device_id=(nbr,), device_id_type=pl.DeviceIdType.MESH,
            )
        pl.semaphore_wait(barrier_sem, 2)

        for s in range(N_DEV - 1):
            if s == 0:
                src = x_ref.at[pl.ds(my * mc, mc), :]
            else:
                src = rs_ref.at[s - 1]
            rdma = pltpu.make_async_remote_copy(
                src_ref=src,
                dst_ref=rs_ref.at[s],
                send_sem=rs_send_sems.at[s],
                recv_sem=rs_recv_sems.at[s],
                device_id=(right,),
                device_id_type=pl.DeviceIdType.MESH,
            )
            rdma.start()
            rdma.wait()
            c = lax.rem(my + 2 * N_DEV - s - 1, N_DEV)
            rs_ref[s, :, :] = rs_ref[s, :, :] + x_ref[pl.ds(c * mc, mc), :]

        owned = lax.rem(my + 1, N_DEV)
        out_ref[pl.ds(owned * mc, mc), :] = rs_ref[N_DEV - 2, :, :]

        for t in range(N_DEV - 1):
            if t == 0:
                src = rs_ref.at[N_DEV - 2]
            else:
                src = ag_ref.at[t - 1]
            rdma = pltpu.make_async_remote_copy(
                src_ref=src,
                dst_ref=ag_ref.at[t],
                send_sem=ag_send_sems.at[t],
                recv_sem=ag_recv_sems.at[t],
                device_id=(right,),
                device_id_type=pl.DeviceIdType.MESH,
            )
            rdma.start()
            rdma.wait()
            owner = lax.rem(my + 2 * N_DEV - t, N_DEV)
            out_ref[pl.ds(owner * mc, mc), :] = ag_ref[t, :, :]

    return pl.pallas_call(
        body,
        out_shape=jax.ShapeDtypeStruct((m, n), x.dtype),
        in_specs=[pl.BlockSpec(memory_space=pltpu.VMEM)],
        out_specs=pl.BlockSpec(memory_space=pltpu.VMEM),
        scratch_shapes=[
            pltpu.VMEM((N_DEV - 1, mc, n), x.dtype),
            pltpu.VMEM((N_DEV - 1, mc, n), x.dtype),
            pltpu.SemaphoreType.DMA((N_DEV - 1,)),
            pltpu.SemaphoreType.DMA((N_DEV - 1,)),
            pltpu.SemaphoreType.DMA((N_DEV - 1,)),
            pltpu.SemaphoreType.DMA((N_DEV - 1,)),
        ],
        compiler_params=pltpu.CompilerParams(
            collective_id=0,
            vmem_limit_bytes=60 * 1024 * 1024,
        ),
    )(x)


# device time: 174579 ns/iter; 1.7689x vs baseline; 1.7689x over previous
import jax
import jax.numpy as jnp
from jax import lax
from jax.experimental import pallas as pl
from jax.experimental.pallas import tpu as pltpu

N_DEV = 4


def kernel(x):
    m, n = x.shape
    mc = m // N_DEV
    hn = n // 2

    def body(x_ref, out_ref, rs_cw, rs_ccw, ag_cw, ag_ccw,
             rs_cw_send, rs_cw_recv, rs_ccw_send, rs_ccw_recv,
             ag_cw_send, ag_cw_recv, ag_ccw_send, ag_ccw_recv):
        my = lax.axis_index("i")
        left = lax.rem(my + N_DEV - 1, N_DEV)
        right = lax.rem(my + 1, N_DEV)

        barrier_sem = pltpu.get_barrier_semaphore()
        for nbr in (left, right):
            pl.semaphore_signal(
                barrier_sem, inc=1,
                device_id=(nbr,), device_id_type=pl.DeviceIdType.MESH,
            )
        pl.semaphore_wait(barrier_sem, 2)

        def start_rdma(src, dst, send_sem, recv_sem, target):
            rdma = pltpu.make_async_remote_copy(
                src_ref=src, dst_ref=dst,
                send_sem=send_sem, recv_sem=recv_sem,
                device_id=(target,), device_id_type=pl.DeviceIdType.MESH,
            )
            rdma.start()
            return rdma

        for s in range(N_DEV - 1):
            if s == 0:
                src_cw = x_ref.at[pl.ds(my * mc, mc), 0:hn]
                src_ccw = x_ref.at[pl.ds(my * mc, mc), hn:n]
            else:
                src_cw = rs_cw.at[s - 1]
                src_ccw = rs_ccw.at[s - 1]
            r_cw = start_rdma(src_cw, rs_cw.at[s],
                              rs_cw_send.at[s], rs_cw_recv.at[s], right)
            r_ccw = start_rdma(src_ccw, rs_ccw.at[s],
                               rs_ccw_send.at[s], rs_ccw_recv.at[s], left)
            r_cw.wait()
            r_ccw.wait()
            c_cw = lax.rem(my + 2 * N_DEV - s - 1, N_DEV)
            c_ccw = lax.rem(my + s + 1, N_DEV)
            rs_cw[s, :, :] = rs_cw[s, :, :] + x_ref[pl.ds(c_cw * mc, mc), 0:hn]
            rs_ccw[s, :, :] = (
                rs_ccw[s, :, :] + x_ref[pl.ds(c_ccw * mc, mc), hn:n]
            )

        own_cw = lax.rem(my + 1, N_DEV)
        own_ccw = lax.rem(my + N_DEV - 1, N_DEV)
        out_ref[pl.ds(own_cw * mc, mc), 0:hn] = rs_cw[N_DEV - 2, :, :]
        out_ref[pl.ds(own_ccw * mc, mc), hn:n] = rs_ccw[N_DEV - 2, :, :]

        for t in range(N_DEV - 1):
            if t == 0:
                src_cw = rs_cw.at[N_DEV - 2]
                src_ccw = rs_ccw.at[N_DEV - 2]
            else:
                src_cw = ag_cw.at[t - 1]
                src_ccw = ag_ccw.at[t - 1]
            r_cw = start_rdma(src_cw, ag_cw.at[t],
                              ag_cw_send.at[t], ag_cw_recv.at[t], right)
            r_ccw = start_rdma(src_ccw, ag_ccw.at[t],
                               ag_ccw_send.at[t], ag_ccw_recv.at[t], left)
            r_cw.wait()
            r_ccw.wait()
            o_cw = lax.rem(my + 2 * N_DEV - t, N_DEV)
            o_ccw = lax.rem(my + t, N_DEV)
            out_ref[pl.ds(o_cw * mc, mc), 0:hn] = ag_cw[t, :, :]
            out_ref[pl.ds(o_ccw * mc, mc), hn:n] = ag_ccw[t, :, :]

    dma3 = pltpu.SemaphoreType.DMA((N_DEV - 1,))
    return pl.pallas_call(
        body,
        out_shape=jax.ShapeDtypeStruct((m, n), x.dtype),
        in_specs=[pl.BlockSpec(memory_space=pltpu.VMEM)],
        out_specs=pl.BlockSpec(memory_space=pltpu.VMEM),
        scratch_shapes=[
            pltpu.VMEM((N_DEV - 1, mc, hn), x.dtype),
            pltpu.VMEM((N_DEV - 1, mc, hn), x.dtype),
            pltpu.VMEM((N_DEV - 1, mc, hn), x.dtype),
            pltpu.VMEM((N_DEV - 1, mc, hn), x.dtype),
            dma3, dma3,
            dma3, dma3,
            dma3, dma3,
            dma3, dma3,
        ],
        compiler_params=pltpu.CompilerParams(
            collective_id=0,
            vmem_limit_bytes=60 * 1024 * 1024,
        ),
    )(x)


# device time: 164576 ns/iter; 1.8765x vs baseline; 1.0608x over previous
import jax
import jax.numpy as jnp
from jax import lax
from jax.experimental import pallas as pl
from jax.experimental.pallas import tpu as pltpu

N_DEV = 4
N_HOP = N_DEV - 1
SUB = 2


def kernel(x):
    m, n = x.shape
    mc = m // N_DEV
    hn = n // 2
    sw = hn // SUB

    def body(x_ref, out_ref, rs_cw, rs_ccw, ag_cw, ag_ccw,
             rs_cw_ss, rs_cw_rs, rs_ccw_ss, rs_ccw_rs,
             ag_cw_ss, ag_cw_rs, ag_ccw_ss, ag_ccw_rs):
        my = lax.axis_index("i")
        left = lax.rem(my + N_DEV - 1, N_DEV)
        right = lax.rem(my + 1, N_DEV)

        barrier_sem = pltpu.get_barrier_semaphore()
        for nbr in (left, right):
            pl.semaphore_signal(
                barrier_sem, inc=1,
                device_id=(nbr,), device_id_type=pl.DeviceIdType.MESH,
            )
        pl.semaphore_wait(barrier_sem, 2)

        def chunk(k):
            return lax.rem(my + k + 2 * N_DEV, N_DEV)

        dirs = [
            (right, 0, rs_cw, ag_cw, rs_cw_ss, rs_cw_rs, ag_cw_ss, ag_cw_rs, -1),
            (left, hn, rs_ccw, ag_ccw, rs_ccw_ss, rs_ccw_rs, ag_ccw_ss, ag_ccw_rs, 1),
        ]

        def start_rdma(src, dst, send_sem, recv_sem, target):
            rdma = pltpu.make_async_remote_copy(
                src_ref=src, dst_ref=dst,
                send_sem=send_sem, recv_sem=recv_sem,
                device_id=(target,), device_id_type=pl.DeviceIdType.MESH,
            )
            rdma.start()
            return rdma

        rs_d = [[[None] * SUB for _ in range(2)] for _ in range(N_HOP)]
        ag_d = [[[None] * SUB for _ in range(2)] for _ in range(N_HOP)]

        for di, (tgt, base, rsb, agb, rss, rsr, agss, agrs, sg) in enumerate(dirs):
            for j in range(SUB):
                rs_d[0][di][j] = start_rdma(
                    x_ref.at[pl.ds(my * mc, mc), base + j * sw:base + (j + 1) * sw],
                    rsb.at[0, :, j * sw:(j + 1) * sw],
                    rss.at[0, j], rsr.at[0, j], tgt,
                )

        for h in range(N_HOP):
            for j in range(SUB):
                for di, (tgt, base, rsb, agb, rss, rsr, agss, agrs, sg) in enumerate(dirs):
                    jb = slice(j * sw, (j + 1) * sw)
                    jx = slice(base + j * sw, base + (j + 1) * sw)
                    rs_d[h][di][j].wait_recv()
                    c = chunk(sg * (h + 1))
                    rsb[h, :, jb] = rsb[h, :, jb] + x_ref[pl.ds(c * mc, mc), jx]
                    if h < N_HOP - 1:
                        rs_d[h + 1][di][j] = start_rdma(
                            rsb.at[h, :, jb], rsb.at[h + 1, :, jb],
                            rss.at[h + 1, j], rsr.at[h + 1, j], tgt,
                        )
                    else:
                        ag_d[0][di][j] = start_rdma(
                            rsb.at[N_HOP - 1, :, jb], agb.at[0, :, jb],
                            agss.at[0, j], agrs.at[0, j], tgt,
                        )

        for di, (tgt, base, rsb, agb, rss, rsr, agss, agrs, sg) in enumerate(dirs):
            out_ref[pl.ds(chunk(-sg) * mc, mc), base:base + hn] = (
                rsb[N_HOP - 1, :, :]
            )

        for t in range(N_HOP):
            for j in range(SUB):
                for di, (tgt, base, rsb, agb, rss, rsr, agss, agrs, sg) in enumerate(dirs):
                    jb = slice(j * sw, (j + 1) * sw)
                    jx = slice(base + j * sw, base + (j + 1) * sw)
                    ag_d[t][di][j].wait_recv()
                    if t < N_HOP - 1:
                        ag_d[t + 1][di][j] = start_rdma(
                            agb.at[t, :, jb], agb.at[t + 1, :, jb],
                            agss.at[t + 1, j], agrs.at[t + 1, j], tgt,
                        )
                    o = chunk(sg * t)
                    out_ref[pl.ds(o * mc, mc), jx] = agb[t, :, jb]

        for dset in (rs_d, ag_d):
            for per_hop in dset:
                for per_dir in per_hop:
                    for r in per_dir:
                        r.wait_send()

    dma_sems = pltpu.SemaphoreType.DMA((N_HOP, SUB))
    return pl.pallas_call(
        body,
        out_shape=jax.ShapeDtypeStruct((m, n), x.dtype),
        in_specs=[pl.BlockSpec(memory_space=pltpu.VMEM)],
        out_specs=pl.BlockSpec(memory_space=pltpu.VMEM),
        scratch_shapes=[
            pltpu.VMEM((N_HOP, mc, hn), x.dtype),
            pltpu.VMEM((N_HOP, mc, hn), x.dtype),
            pltpu.VMEM((N_HOP, mc, hn), x.dtype),
            pltpu.VMEM((N_HOP, mc, hn), x.dtype),
            dma_sems, dma_sems,
            dma_sems, dma_sems,
            dma_sems, dma_sems,
            dma_sems, dma_sems,
        ],
        compiler_params=pltpu.CompilerParams(
            collective_id=0,
            vmem_limit_bytes=60 * 1024 * 1024,
        ),
    )(x)


# device time: 164397 ns/iter; 1.8785x vs baseline; 1.0011x over previous
import jax
import jax.numpy as jnp
from jax import lax
from jax.experimental import pallas as pl
from jax.experimental.pallas import tpu as pltpu

N_DEV = 4
N_HOP = N_DEV - 1
SUB = 2


def kernel(x):
    m, n = x.shape
    mc = m // N_DEV
    hr = mc // 2
    sr = hr // SUB

    def body(x_ref, out_ref, rs_cw, rs_ccw, ag_cw, ag_ccw,
             rs_cw_ss, rs_cw_rs, rs_ccw_ss, rs_ccw_rs,
             ag_cw_ss, ag_cw_rs, ag_ccw_ss, ag_ccw_rs):
        my = lax.axis_index("i")
        left = lax.rem(my + N_DEV - 1, N_DEV)
        right = lax.rem(my + 1, N_DEV)

        barrier_sem = pltpu.get_barrier_semaphore()
        for nbr in (left, right):
            pl.semaphore_signal(
                barrier_sem, inc=1,
                device_id=(nbr,), device_id_type=pl.DeviceIdType.MESH,
            )
        pl.semaphore_wait(barrier_sem, 2)

        def chunk(k):
            return lax.rem(my + k + 2 * N_DEV, N_DEV)

        dirs = [
            (right, 0, rs_cw, ag_cw, rs_cw_ss, rs_cw_rs, ag_cw_ss, ag_cw_rs, -1),
            (left, hr, rs_ccw, ag_ccw, rs_ccw_ss, rs_ccw_rs, ag_ccw_ss, ag_ccw_rs, 1),
        ]

        def start_rdma(src, dst, send_sem, recv_sem, target):
            rdma = pltpu.make_async_remote_copy(
                src_ref=src, dst_ref=dst,
                send_sem=send_sem, recv_sem=recv_sem,
                device_id=(target,), device_id_type=pl.DeviceIdType.MESH,
            )
            rdma.start()
            return rdma

        rs_d = [[[None] * SUB for _ in range(2)] for _ in range(N_HOP)]
        ag_d = [[[None] * SUB for _ in range(2)] for _ in range(N_HOP)]

        for di, (tgt, base, rsb, agb, rss, rsr, agss, agrs, sg) in enumerate(dirs):
            for j in range(SUB):
                rs_d[0][di][j] = start_rdma(
                    x_ref.at[pl.ds(my * mc + base + j * sr, sr), :],
                    rsb.at[0, j * sr:(j + 1) * sr, :],
                    rss.at[0, j], rsr.at[0, j], tgt,
                )

        for h in range(N_HOP):
            for j in range(SUB):
                for di, (tgt, base, rsb, agb, rss, rsr, agss, agrs, sg) in enumerate(dirs):
                    jb = slice(j * sr, (j + 1) * sr)
                    rs_d[h][di][j].wait_recv()
                    c = chunk(sg * (h + 1))
                    rsb[h, jb, :] = (
                        rsb[h, jb, :]
                        + x_ref[pl.ds(c * mc + base + j * sr, sr), :]
                    )
                    if h < N_HOP - 1:
                        rs_d[h + 1][di][j] = start_rdma(
                            rsb.at[h, jb, :], rsb.at[h + 1, jb, :],
                            rss.at[h + 1, j], rsr.at[h + 1, j], tgt,
                        )
                    else:
                        ag_d[0][di][j] = start_rdma(
                            rsb.at[N_HOP - 1, jb, :], agb.at[0, jb, :],
                            agss.at[0, j], agrs.at[0, j], tgt,
                        )

        for di, (tgt, base, rsb, agb, rss, rsr, agss, agrs, sg) in enumerate(dirs):
            out_ref[pl.ds(chunk(-sg) * mc + base, hr), :] = rsb[N_HOP - 1, :, :]

        for t in range(N_HOP):
            for j in range(SUB):
                for di, (tgt, base, rsb, agb, rss, rsr, agss, agrs, sg) in enumerate(dirs):
                    jb = slice(j * sr, (j + 1) * sr)
                    ag_d[t][di][j].wait_recv()
                    if t < N_HOP - 1:
                        ag_d[t + 1][di][j] = start_rdma(
                            agb.at[t, jb, :], agb.at[t + 1, jb, :],
                            agss.at[t + 1, j], agrs.at[t + 1, j], tgt,
                        )
                    o = chunk(sg * t)
                    out_ref[pl.ds(o * mc + base + j * sr, sr), :] = agb[t, jb, :]

        for dset in (rs_d, ag_d):
            for per_hop in dset:
                for per_dir in per_hop:
                    for r in per_dir:
                        r.wait_send()

    dma_sems = pltpu.SemaphoreType.DMA((N_HOP, SUB))
    return pl.pallas_call(
        body,
        out_shape=jax.ShapeDtypeStruct((m, n), x.dtype),
        in_specs=[pl.BlockSpec(memory_space=pltpu.VMEM)],
        out_specs=pl.BlockSpec(memory_space=pltpu.VMEM),
        scratch_shapes=[
            pltpu.VMEM((N_HOP, hr, n), x.dtype),
            pltpu.VMEM((N_HOP, hr, n), x.dtype),
            pltpu.VMEM((N_HOP, hr, n), x.dtype),
            pltpu.VMEM((N_HOP, hr, n), x.dtype),
            dma_sems, dma_sems,
            dma_sems, dma_sems,
            dma_sems, dma_sems,
            dma_sems, dma_sems,
        ],
        compiler_params=pltpu.CompilerParams(
            collective_id=0,
            vmem_limit_bytes=60 * 1024 * 1024,
        ),
    )(x)


# device time: 159963 ns/iter; 1.9306x vs baseline; 1.0277x over previous
import jax
import jax.numpy as jnp
from jax import lax
from jax.experimental import pallas as pl
from jax.experimental.pallas import tpu as pltpu

N_DEV = 4
N_HOP = N_DEV - 1
SUB = 2


def kernel(x):
    m, n = x.shape
    mc = m // N_DEV
    hr = mc // 2
    sr = hr // SUB

    def body(x_ref, out_ref, rs_cw, rs_ccw, ag_cw, ag_ccw,
             rs_cw_ss, rs_cw_rs, rs_ccw_ss, rs_ccw_rs,
             ag_cw_ss, ag_cw_rs, ag_ccw_ss, ag_ccw_rs, st_sems):
        my = lax.axis_index("i")
        left = lax.rem(my + N_DEV - 1, N_DEV)
        right = lax.rem(my + 1, N_DEV)

        barrier_sem = pltpu.get_barrier_semaphore()
        for nbr in (left, right):
            pl.semaphore_signal(
                barrier_sem, inc=1,
                device_id=(nbr,), device_id_type=pl.DeviceIdType.MESH,
            )
        pl.semaphore_wait(barrier_sem, 2)

        def chunk(k):
            return lax.rem(my + k + 2 * N_DEV, N_DEV)

        dirs = [
            (right, 0, rs_cw, ag_cw, rs_cw_ss, rs_cw_rs, ag_cw_ss, ag_cw_rs, -1),
            (left, hr, rs_ccw, ag_ccw, rs_ccw_ss, rs_ccw_rs, ag_ccw_ss, ag_ccw_rs, 1),
        ]

        def start_rdma(src, dst, send_sem, recv_sem, target):
            rdma = pltpu.make_async_remote_copy(
                src_ref=src, dst_ref=dst,
                send_sem=send_sem, recv_sem=recv_sem,
                device_id=(target,), device_id_type=pl.DeviceIdType.MESH,
            )
            rdma.start()
            return rdma

        stores = []

        def store_out(src, dst_rows, nrows):
            cp = pltpu.make_async_copy(
                src, out_ref.at[pl.ds(dst_rows, nrows), :],
                st_sems.at[len(stores)],
            )
            cp.start()
            stores.append(cp)

        rs_d = [[[None] * SUB for _ in range(2)] for _ in range(N_HOP)]
        ag_d = [[[None] * SUB for _ in range(2)] for _ in range(N_HOP)]

        for di, (tgt, base, rsb, agb, rss, rsr, agss, agrs, sg) in enumerate(dirs):
            for j in range(SUB):
                rs_d[0][di][j] = start_rdma(
                    x_ref.at[pl.ds(my * mc + base + j * sr, sr), :],
                    rsb.at[0, j * sr:(j + 1) * sr, :],
                    rss.at[0, j], rsr.at[0, j], tgt,
                )

        for h in range(N_HOP):
            for j in range(SUB):
                for di, (tgt, base, rsb, agb, rss, rsr, agss, agrs, sg) in enumerate(dirs):
                    jb = slice(j * sr, (j + 1) * sr)
                    rs_d[h][di][j].wait_recv()
                    c = chunk(sg * (h + 1))
                    rsb[h, jb, :] = (
                        rsb[h, jb, :]
                        + x_ref[pl.ds(c * mc + base + j * sr, sr), :]
                    )
                    if h < N_HOP - 1:
                        rs_d[h + 1][di][j] = start_rdma(
                            rsb.at[h, jb, :], rsb.at[h + 1, jb, :],
                            rss.at[h + 1, j], rsr.at[h + 1, j], tgt,
                        )
                    else:
                        ag_d[0][di][j] = start_rdma(
                            rsb.at[N_HOP - 1, jb, :], agb.at[0, jb, :],
                            agss.at[0, j], agrs.at[0, j], tgt,
                        )
                        store_out(rsb.at[N_HOP - 1, jb, :],
                                  chunk(-sg) * mc + base + j * sr, sr)

        for t in range(N_HOP):
            for j in range(SUB):
                for di, (tgt, base, rsb, agb, rss, rsr, agss, agrs, sg) in enumerate(dirs):
                    jb = slice(j * sr, (j + 1) * sr)
                    ag_d[t][di][j].wait_recv()
                    if t < N_HOP - 1:
                        ag_d[t + 1][di][j] = start_rdma(
                            agb.at[t, jb, :], agb.at[t + 1, jb, :],
                            agss.at[t + 1, j], agrs.at[t + 1, j], tgt,
                        )
                    o = chunk(sg * t)
                    store_out(agb.at[t, jb, :],
                              o * mc + base + j * sr, sr)

        for cp in stores:
            cp.wait()
        for dset in (rs_d, ag_d):
            for per_hop in dset:
                for per_dir in per_hop:
                    for r in per_dir:
                        r.wait_send()

    n_stores = 2 * SUB + N_HOP * 2 * SUB
    dma_sems = pltpu.SemaphoreType.DMA((N_HOP, SUB))
    return pl.pallas_call(
        body,
        out_shape=jax.ShapeDtypeStruct((m, n), x.dtype),
        in_specs=[pl.BlockSpec(memory_space=pltpu.VMEM)],
        out_specs=pl.BlockSpec(memory_space=pl.ANY),
        scratch_shapes=[
            pltpu.VMEM((N_HOP, hr, n), x.dtype),
            pltpu.VMEM((N_HOP, hr, n), x.dtype),
            pltpu.VMEM((N_HOP, hr, n), x.dtype),
            pltpu.VMEM((N_HOP, hr, n), x.dtype),
            dma_sems, dma_sems,
            dma_sems, dma_sems,
            dma_sems, dma_sems,
            dma_sems, dma_sems,
            pltpu.SemaphoreType.DMA((n_stores,)),
        ],
        compiler_params=pltpu.CompilerParams(
            collective_id=0,
            vmem_limit_bytes=60 * 1024 * 1024,
        ),
    )(x)
